# baseline (speedup 1.0000x reference)
import jax
import jax.numpy as jnp
from jax.experimental import pallas as pl
from jax.experimental.pallas import tpu as pltpu


def _dist_kernel(x_ref, w_ref, o_ref):
    x = x_ref[...]
    w = w_ref[...]
    bi, gj, d = w.shape
    b = x.shape[0]
    w2 = w.reshape(bi * gj, d)
    xn = jnp.sum(x * x, axis=1, keepdims=True)
    wn = jnp.sum(w2 * w2, axis=1, keepdims=True)
    xa = jnp.concatenate(
        [x * -2.0, xn, jnp.ones((b, 1), jnp.float32)], axis=1)
    wa = jnp.concatenate(
        [w2, jnp.ones((bi * gj, 1), jnp.float32), wn], axis=1)
    r = jax.lax.dot_general(
        xa, wa, (((1,), (1,)), ((), ())),
        preferred_element_type=jnp.float32,
        precision=jax.lax.Precision.DEFAULT,
    )
    o_ref[...] = r.reshape(b, bi, gj)


def _dist_kernel_b(x_ref, w_ref, o_ref):
    x = x_ref[...]
    w = w_ref[...]
    g0, g1, d = w.shape
    b = x.shape[0]
    w2 = w.reshape(g0 * g1, d)
    xn = jnp.sum(x * x, axis=1, keepdims=True)
    wn = jnp.sum(w2 * w2, axis=1, keepdims=True)
    xa = jnp.concatenate(
        [x * -2.0, xn, jnp.ones((b, 1), jnp.float32)], axis=1)
    wa = jnp.concatenate(
        [w2, jnp.ones((g0 * g1, 1), jnp.float32), wn], axis=1)
    r = jax.lax.dot_general(
        xa, wa, (((1,), (1,)), ((), ())),
        preferred_element_type=jnp.float32,
        precision=jax.lax.Precision.DEFAULT,
    )
    o_ref[...] = r.reshape(b, g0, g1)


def kernel(x, weights):
    B, D = x.shape
    G0, G1, _ = weights.shape
    bb = 128
    out = pl.pallas_call(
        _dist_kernel_b,
        grid=(B // bb,),
        in_specs=[
            pl.BlockSpec((bb, D), lambda g: (g, 0)),
            pl.BlockSpec((G0, G1, D), lambda g: (0, 0, 0)),
        ],
        out_specs=pl.BlockSpec((bb, G0, G1), lambda g: (g, 0, 0)),
        out_shape=jax.ShapeDtypeStruct((B, G0, G1), jnp.float32),
        compiler_params=pltpu.CompilerParams(
            dimension_semantics=("parallel",)),
    )(x, weights)
    return out


# speedup vs baseline: 1.1412x; 1.1412x over previous
import jax
import jax.numpy as jnp
from jax.experimental import pallas as pl
from jax.experimental.pallas import tpu as pltpu


def _dist_kernel(x_ref, w_ref, o_ref):
    x = x_ref[...]
    w = w_ref[...]
    bi, gj, d = w.shape
    b = x.shape[0]
    w2 = w.reshape(bi * gj, d)
    xn = jnp.sum(x * x, axis=1, keepdims=True)
    wn = jnp.sum(w2 * w2, axis=1, keepdims=True)
    xa = jnp.concatenate(
        [x * -2.0, xn, jnp.ones((b, 1), jnp.float32)], axis=1)
    wa = jnp.concatenate(
        [w2, jnp.ones((bi * gj, 1), jnp.float32), wn], axis=1)
    r = jax.lax.dot_general(
        xa, wa, (((1,), (1,)), ((), ())),
        preferred_element_type=jnp.float32,
        precision=jax.lax.Precision.DEFAULT,
    )
    o_ref[...] = r.reshape(b, bi, gj)


def kernel(x, weights):
    B, D = x.shape
    G0, G1, _ = weights.shape
    bi = 32
    out = pl.pallas_call(
        _dist_kernel,
        grid=(G0 // bi,),
        in_specs=[
            pl.BlockSpec((B, D), lambda g: (0, 0)),
            pl.BlockSpec((bi, G1, D), lambda g: (g, 0, 0)),
        ],
        out_specs=pl.BlockSpec((B, bi, G1), lambda g: (0, g, 0)),
        out_shape=jax.ShapeDtypeStruct((B, G0, G1), jnp.float32),
        compiler_params=pltpu.CompilerParams(
            dimension_semantics=("parallel",)),
    )(x, weights)
    return out
